# baseline (speedup 1.0000x reference)
import functools

import jax
import jax.numpy as jnp
from jax import lax
from jax.experimental import pallas as pl
from jax.experimental.pallas import tpu as pltpu
from jax.experimental.pallas import tpu_sc as plsc

N = 10000
E = 320000
F_IN = 128
HID = 128
EMB = 64
N_IDX = 2048

HE_W = 136
NP = 10240
NWORK = 32
EPW = E // NWORK
KC = 80
NCH = EPW // KC
NB = 5
CPB = NCH // NB
RPS = NP // 16
IPW = N_IDX // NWORK

@functools.cache
def _mesh():
    return plsc.VectorSubcoreMesh(
        core_axis_name="c", subcore_axis_name="s", num_cores=2, num_subcores=16)


_BLK = 1024


def _enc_tail(h, s, he_ref, sp_ref):
    lane = lax.broadcasted_iota(jnp.int32, (_BLK, HE_W - HID), 1)
    extra = jnp.where(lane == 0, s[:, 1:2], 0.0)
    he_ref[...] = jnp.concatenate([h, extra], axis=1)
    sp_ref[...] = s[:, 0:1]


def _encode_body(x_ref, w_ref, a_ref, he_ref, sp_ref):
    h = jnp.dot(x_ref[...], w_ref[...], preferred_element_type=jnp.float32)
    s = jnp.dot(h, a_ref[...], preferred_element_type=jnp.float32)
    _enc_tail(h, s, he_ref, sp_ref)


def _agg(acc_ref, b_ref, sc_ref, sh_ref):
    a = acc_ref[0] + acc_ref[1]
    x = a[:, :HID] / jnp.maximum(a[:, HID:HID + 1], 1e-9) + b_ref[...]
    return jnp.maximum(x * sc_ref[...] + sh_ref[...], 0.0)


def _agg_encode_body(acc_ref, b_ref, sc_ref, sh_ref, w_ref, a_ref, he_ref, sp_ref):
    x = _agg(acc_ref, b_ref, sc_ref, sh_ref)
    h = jnp.dot(x, w_ref[...], preferred_element_type=jnp.float32)
    s = jnp.dot(h, a_ref[...], preferred_element_type=jnp.float32)
    _enc_tail(h, s, he_ref, sp_ref)


def _agg_dense_body(acc_ref, b_ref, sc_ref, sh_ref, wd_ref, bd_ref, y_ref):
    x = _agg(acc_ref, b_ref, sc_ref, sh_ref)
    y_ref[...] = (
        jnp.dot(x, wd_ref[...], preferred_element_type=jnp.float32) + bd_ref[...])


def _vec_spec():
    return pl.BlockSpec((1, HID), lambda i: (0, 0))


_encode = pl.pallas_call(
    _encode_body,
    grid=(NP // _BLK,),
    in_specs=[
        pl.BlockSpec((_BLK, F_IN), lambda i: (i, 0)),
        pl.BlockSpec((F_IN, HID), lambda i: (0, 0)),
        pl.BlockSpec((HID, 16), lambda i: (0, 0)),
    ],
    out_specs=[
        pl.BlockSpec((_BLK, HE_W), lambda i: (i, 0)),
        pl.BlockSpec((_BLK, 1), lambda i: (i, 0)),
    ],
    out_shape=[
        jax.ShapeDtypeStruct((NP, HE_W), jnp.float32),
        jax.ShapeDtypeStruct((NP, 1), jnp.float32),
    ],
)

_agg_encode = pl.pallas_call(
    _agg_encode_body,
    grid=(NP // _BLK,),
    in_specs=[
        pl.BlockSpec((2, _BLK, HE_W), lambda i: (0, i, 0)),
        _vec_spec(), _vec_spec(), _vec_spec(),
        pl.BlockSpec((HID, HID), lambda i: (0, 0)),
        pl.BlockSpec((HID, 16), lambda i: (0, 0)),
    ],
    out_specs=[
        pl.BlockSpec((_BLK, HE_W), lambda i: (i, 0)),
        pl.BlockSpec((_BLK, 1), lambda i: (i, 0)),
    ],
    out_shape=[
        jax.ShapeDtypeStruct((NP, HE_W), jnp.float32),
        jax.ShapeDtypeStruct((NP, 1), jnp.float32),
    ],
)

_agg_dense = pl.pallas_call(
    _agg_dense_body,
    grid=(NP // _BLK,),
    in_specs=[
        pl.BlockSpec((2, _BLK, HE_W), lambda i: (0, i, 0)),
        _vec_spec(), _vec_spec(), _vec_spec(),
        pl.BlockSpec((HID, EMB), lambda i: (0, 0)),
        pl.BlockSpec((1, EMB), lambda i: (0, 0)),
    ],
    out_specs=pl.BlockSpec((_BLK, EMB), lambda i: (i, 0)),
    out_shape=jax.ShapeDtypeStruct((NP, EMB), jnp.float32),
)


def _gat_sc_body(he_hbm, sp_hbm, ei_hbm, ej_hbm, out_hbm,
                 ei_a, ej_a, ei_b, ej_b, sp_vm, rows_a, rows_b, w_vm, acc_sh,
                 gsa, gsb, isa, isb, ssa, ssb):
    cid = lax.axis_index("c")
    sid = lax.axis_index("s")
    wid = cid * 16 + sid

    pltpu.sync_copy(sp_hbm, sp_vm)

    lane16 = lax.iota(jnp.int32, 16)

    def _zr(r, carry):
        z = jnp.zeros((16,), jnp.float32)
        for c in range(HID // 16):
            rows_a[r, pl.ds(c * 16, 16)] = z
        plsc.store_scatter(rows_a, [jnp.full((16,), r, jnp.int32), HID + lane16],
                           z, mask=lane16 < HE_W - HID)
        return carry
    lax.fori_loop(0, KC, _zr, 0)
    base = sid * RPS
    for k in range(RPS // KC):
        pltpu.sync_copy(rows_a, acc_sh.at[pl.ds(base + k * KC, KC)])
    plsc.subcore_barrier()

    def _process(ci, eib, buf):
        for g in range(KC // 16):
            eiv = eib[ci, 0, pl.ds(g * 16, 16)]
            spv = plsc.load_gather(sp_vm, [eiv])
            ridx = lane16 + g * 16
            sqv = plsc.load_gather(buf, [ridx, jnp.full((16,), HID, jnp.int32)])
            ev = spv + sqv
            ev = jnp.where(ev > 0, ev, ev * 0.2)
            w_vm[pl.ds(g * 16, 16)] = jnp.exp(ev)

        def _srow(r, c2):
            wv = plsc.load_gather(w_vm, [jnp.full((16,), r, jnp.int32)])
            for c in range(HID // 16):
                buf[r, pl.ds(c * 16, 16)] = buf[r, pl.ds(c * 16, 16)] * wv
            plsc.store_scatter(buf, [jnp.full((16,), r, jnp.int32), HID + lane16],
                               wv, mask=lane16 < HE_W - HID)
            return c2
        lax.fori_loop(0, KC, _srow, 0)

    idx_bufs = [(ei_a, ej_a, isa), (ei_b, ej_b, isb)]

    def _idx_start(b, eib, ejb, isem):
        pltpu.async_copy(ei_hbm.at[wid, pl.ds(b * CPB, CPB)], eib, isem)
        pltpu.async_copy(ej_hbm.at[wid, pl.ds(b * CPB, CPB)], ejb, isem)

    def _idx_wait(b, eib, ejb, isem):
        pltpu.make_async_copy(ei_hbm.at[wid, pl.ds(b * CPB, CPB)], eib, isem).wait()
        pltpu.make_async_copy(ej_hbm.at[wid, pl.ds(b * CPB, CPB)], ejb, isem).wait()

    _idx_start(0, ei_a, ej_a, isa)
    _idx_start(1, ei_b, ej_b, isb)

    for b in range(NB):
        eib, ejb, isem = idx_bufs[b % 2]
        _idx_wait(b, eib, ejb, isem)

        def _gs(ci, buf, gsem):
            pltpu.async_copy(he_hbm.at[ejb.at[ci, 0]], buf, gsem)

        def _gw(ci, buf, gsem):
            pltpu.make_async_copy(he_hbm.at[ejb.at[ci, 0]], buf, gsem).wait()

        def _ss(ci, buf, ssem):
            pltpu.async_copy(buf, acc_sh.at[eib.at[ci, 0]], ssem, add=True)

        def _sw(ci, buf, ssem):
            pltpu.make_async_copy(buf, acc_sh.at[eib.at[ci, 0]], ssem).wait()

        _gs(0, rows_a, gsa)
        _gs(1, rows_b, gsb)
        _gw(0, rows_a, gsa)
        _process(0, eib, rows_a)
        _ss(0, rows_a, ssa)

        def _pbody(o, prefetch):
            _gw(o, rows_b, gsb)
            _process(o, eib, rows_b)
            _sw(o - 1, rows_a, ssa)
            _gs(o + 1, rows_a, gsa)
            _ss(o, rows_b, ssb)
            _gw(o + 1, rows_a, gsa)
            _process(o + 1, eib, rows_a)
            _sw(o, rows_b, ssb)
            if prefetch:
                _gs(o + 2, rows_b, gsb)
            _ss(o + 1, rows_a, ssa)

        def _pair(p, carry):
            _pbody(2 * p + 1, True)
            return carry
        lax.fori_loop(0, (CPB - 3) // 2, _pair, 0)
        _pbody(CPB - 2, False)
        _sw(CPB - 1, rows_a, ssa)

        if b + 2 < NB:
            _idx_start(b + 2, eib, ejb, isem)

    plsc.subcore_barrier()
    pltpu.sync_copy(acc_sh.at[pl.ds(base, RPS)], out_hbm.at[cid, pl.ds(base, RPS)])


@functools.cache
def _gat_sc():
    return pl.kernel(
        _gat_sc_body,
        out_type=jax.ShapeDtypeStruct((2, NP, HE_W), jnp.float32),
        mesh=_mesh(),
        compiler_params=pltpu.CompilerParams(needs_layout_passes=False, use_tc_tiling_on_sc=False),
        scratch_types=[
            pltpu.VMEM((CPB, 1, KC), jnp.int32),
            pltpu.VMEM((CPB, 1, KC), jnp.int32),
            pltpu.VMEM((CPB, 1, KC), jnp.int32),
            pltpu.VMEM((CPB, 1, KC), jnp.int32),
            pltpu.VMEM((NP,), jnp.float32),
            pltpu.VMEM((KC, HE_W), jnp.float32),
            pltpu.VMEM((KC, HE_W), jnp.float32),
            pltpu.VMEM((KC,), jnp.float32),
            pltpu.VMEM_SHARED((NP, HE_W), jnp.float32),
            pltpu.SemaphoreType.DMA,
            pltpu.SemaphoreType.DMA,
            pltpu.SemaphoreType.DMA,
            pltpu.SemaphoreType.DMA,
            pltpu.SemaphoreType.DMA,
            pltpu.SemaphoreType.DMA,
        ],
    )


def _take_sc_body(y_hbm, idx_hbm, out_hbm, idx_vm, rows_vm, sem):
    cid = lax.axis_index("c")
    sid = lax.axis_index("s")
    base = (cid * 16 + sid) * IPW
    pltpu.sync_copy(idx_hbm.at[pl.ds(base, IPW)], idx_vm)
    pltpu.async_copy(y_hbm.at[idx_vm], rows_vm, sem).wait()
    pltpu.sync_copy(rows_vm, out_hbm.at[pl.ds(base, IPW)])


@functools.cache
def _take_sc():
    return pl.kernel(
        _take_sc_body,
        out_type=jax.ShapeDtypeStruct((N_IDX, EMB), jnp.float32),
        mesh=_mesh(),
        compiler_params=pltpu.CompilerParams(needs_layout_passes=False, use_tc_tiling_on_sc=False),
        scratch_types=[
            pltpu.VMEM((IPW,), jnp.int32),
            pltpu.VMEM((IPW, EMB), jnp.float32),
            pltpu.SemaphoreType.DMA,
        ],
    )


def _a_pad(a1, a2):
    a = jnp.zeros((HID, 16), jnp.float32)
    return a.at[:, 0].set(a1).at[:, 1].set(a2)


def _bn_consts(gamma, beta, mean, var):
    sc = gamma / jnp.sqrt(var + 1e-5)
    sh = beta - mean * sc
    return sc.reshape(1, HID), sh.reshape(1, HID)


def kernel(features, edge_index, idx, W0, a1_0, a2_0, b0, gamma0, beta0,
           mean0, var0, W1, a1_1, a2_1, b1, gamma1, beta1, mean1, var1, Wd, bd):
    ei = edge_index[0].reshape(NWORK, NCH, 1, KC)
    ej = edge_index[1].reshape(NWORK, NCH, 1, KC)

    sc0, sh0 = _bn_consts(gamma0, beta0, mean0, var0)
    sc1, sh1 = _bn_consts(gamma1, beta1, mean1, var1)

    xp = jnp.pad(features, ((0, NP - N), (0, 0)))
    he0, sp0 = _encode(xp, W0, _a_pad(a1_0, a2_0))
    acc0 = _gat_sc()(he0, sp0.reshape(NP), ei, ej)
    he1, sp1 = _agg_encode(acc0, b0.reshape(1, HID), sc0, sh0, W1,
                           _a_pad(a1_1, a2_1))
    acc1 = _gat_sc()(he1, sp1.reshape(NP), ei, ej)
    y = _agg_dense(acc1, b1.reshape(1, HID), sc1, sh1, Wd, bd.reshape(1, EMB))
    return _take_sc()(y, idx)


# speedup vs baseline: 1.0005x; 1.0005x over previous
import functools

import jax
import jax.numpy as jnp
from jax import lax
from jax.experimental import pallas as pl
from jax.experimental.pallas import tpu as pltpu
from jax.experimental.pallas import tpu_sc as plsc

N = 10000
E = 320000
F_IN = 128
HID = 128
EMB = 64
N_IDX = 2048

HE_W = 136
NP = 10240
NWORK = 32
EPW = E // NWORK
KC = 80
NCH = EPW // KC
NB = 5
CPB = NCH // NB
RPS = NP // 16
IPW = N_IDX // NWORK

@functools.cache
def _mesh():
    return plsc.VectorSubcoreMesh(
        core_axis_name="c", subcore_axis_name="s", num_cores=2, num_subcores=16)


_BLK = 1024


def _enc_tail(h, s, he_ref, sp_ref):
    lane = lax.broadcasted_iota(jnp.int32, (_BLK, HE_W - HID), 1)
    extra = jnp.where(lane == 0, s[:, 1:2], 0.0)
    he_ref[...] = jnp.concatenate([h, extra], axis=1)
    sp_ref[...] = s[:, 0:1]


def _encode_body(x_ref, w_ref, a_ref, he_ref, sp_ref):
    h = jnp.dot(x_ref[...], w_ref[...], preferred_element_type=jnp.float32)
    s = jnp.dot(h, a_ref[...], preferred_element_type=jnp.float32)
    _enc_tail(h, s, he_ref, sp_ref)


def _agg(acc_ref, b_ref, sc_ref, sh_ref):
    a = acc_ref[0] + acc_ref[1]
    x = a[:, :HID] / jnp.maximum(a[:, HID:HID + 1], 1e-9) + b_ref[...]
    return jnp.maximum(x * sc_ref[...] + sh_ref[...], 0.0)


def _agg_encode_body(acc_ref, b_ref, sc_ref, sh_ref, w_ref, a_ref, he_ref, sp_ref):
    x = _agg(acc_ref, b_ref, sc_ref, sh_ref)
    h = jnp.dot(x, w_ref[...], preferred_element_type=jnp.float32)
    s = jnp.dot(h, a_ref[...], preferred_element_type=jnp.float32)
    _enc_tail(h, s, he_ref, sp_ref)


def _agg_dense_body(acc_ref, b_ref, sc_ref, sh_ref, wd_ref, bd_ref, y_ref):
    x = _agg(acc_ref, b_ref, sc_ref, sh_ref)
    y_ref[...] = (
        jnp.dot(x, wd_ref[...], preferred_element_type=jnp.float32) + bd_ref[...])


def _vec_spec():
    return pl.BlockSpec((1, HID), lambda i: (0, 0))


_encode = pl.pallas_call(
    _encode_body,
    grid=(NP // _BLK,),
    in_specs=[
        pl.BlockSpec((_BLK, F_IN), lambda i: (i, 0)),
        pl.BlockSpec((F_IN, HID), lambda i: (0, 0)),
        pl.BlockSpec((HID, 16), lambda i: (0, 0)),
    ],
    out_specs=[
        pl.BlockSpec((_BLK, HE_W), lambda i: (i, 0)),
        pl.BlockSpec((_BLK, 1), lambda i: (i, 0)),
    ],
    out_shape=[
        jax.ShapeDtypeStruct((NP, HE_W), jnp.float32),
        jax.ShapeDtypeStruct((NP, 1), jnp.float32),
    ],
)

_agg_encode = pl.pallas_call(
    _agg_encode_body,
    grid=(NP // _BLK,),
    in_specs=[
        pl.BlockSpec((2, _BLK, HE_W), lambda i: (0, i, 0)),
        _vec_spec(), _vec_spec(), _vec_spec(),
        pl.BlockSpec((HID, HID), lambda i: (0, 0)),
        pl.BlockSpec((HID, 16), lambda i: (0, 0)),
    ],
    out_specs=[
        pl.BlockSpec((_BLK, HE_W), lambda i: (i, 0)),
        pl.BlockSpec((_BLK, 1), lambda i: (i, 0)),
    ],
    out_shape=[
        jax.ShapeDtypeStruct((NP, HE_W), jnp.float32),
        jax.ShapeDtypeStruct((NP, 1), jnp.float32),
    ],
)

_agg_dense = pl.pallas_call(
    _agg_dense_body,
    grid=(NP // _BLK,),
    in_specs=[
        pl.BlockSpec((2, _BLK, HE_W), lambda i: (0, i, 0)),
        _vec_spec(), _vec_spec(), _vec_spec(),
        pl.BlockSpec((HID, EMB), lambda i: (0, 0)),
        pl.BlockSpec((1, EMB), lambda i: (0, 0)),
    ],
    out_specs=pl.BlockSpec((_BLK, EMB), lambda i: (i, 0)),
    out_shape=jax.ShapeDtypeStruct((NP, EMB), jnp.float32),
)


def _gat_sc_body(he_hbm, sp_hbm, ei_hbm, ej_hbm, out_hbm,
                 ei_a, ej_a, ei_b, ej_b, sp_vm, rows_a, rows_b, w_vm, acc_sh,
                 gsa, gsb, isa, isb, ssa, ssb):
    cid = lax.axis_index("c")
    sid = lax.axis_index("s")
    wid = cid * 16 + sid

    pltpu.sync_copy(sp_hbm, sp_vm)

    lane16 = lax.iota(jnp.int32, 16)

    def _zr(r, carry):
        z = jnp.zeros((16,), jnp.float32)
        for c in range(HID // 16):
            rows_a[r, pl.ds(c * 16, 16)] = z
        plsc.store_scatter(rows_a, [jnp.full((16,), r, jnp.int32), HID + lane16],
                           z, mask=lane16 < HE_W - HID)
        return carry
    lax.fori_loop(0, KC, _zr, 0)
    base = sid * RPS
    for k in range(RPS // KC):
        pltpu.sync_copy(rows_a, acc_sh.at[pl.ds(base + k * KC, KC)])
    plsc.subcore_barrier()

    def _process(ci, eib, buf):
        for g in range(KC // 16):
            eiv = eib[ci, 0, pl.ds(g * 16, 16)]
            spv = plsc.load_gather(sp_vm, [eiv])
            ridx = lane16 + g * 16
            sqv = plsc.load_gather(buf, [ridx, jnp.full((16,), HID, jnp.int32)])
            ev = spv + sqv
            ev = jnp.where(ev > 0, ev, ev * 0.2)
            w_vm[pl.ds(g * 16, 16)] = jnp.exp(ev)

        def _srow(r, c2):
            wv = plsc.load_gather(w_vm, [jnp.full((16,), r, jnp.int32)])
            for c in range(HID // 16):
                buf[r, pl.ds(c * 16, 16)] = buf[r, pl.ds(c * 16, 16)] * wv
            plsc.store_scatter(buf, [jnp.full((16,), r, jnp.int32), HID + lane16],
                               wv, mask=lane16 < HE_W - HID)
            return c2
        lax.fori_loop(0, KC, _srow, 0)

    idx_bufs = [(ei_a, ej_a, isa), (ei_b, ej_b, isb)]

    def _idx_start(b, eib, ejb, isem):
        pltpu.async_copy(ei_hbm.at[wid, pl.ds(b * CPB, CPB)], eib, isem)
        pltpu.async_copy(ej_hbm.at[wid, pl.ds(b * CPB, CPB)], ejb, isem)

    def _idx_wait(b, eib, ejb, isem):
        pltpu.make_async_copy(ei_hbm.at[wid, pl.ds(b * CPB, CPB)], eib, isem).wait()
        pltpu.make_async_copy(ej_hbm.at[wid, pl.ds(b * CPB, CPB)], ejb, isem).wait()

    _idx_start(0, ei_a, ej_a, isa)
    _idx_start(1, ei_b, ej_b, isb)

    for b in range(NB):
        eib, ejb, isem = idx_bufs[b % 2]
        _idx_wait(b, eib, ejb, isem)

        def _gs(ci, buf, gsem):
            pltpu.async_copy(he_hbm.at[ejb.at[ci, 0]], buf, gsem)

        def _gw(ci, buf, gsem):
            pltpu.make_async_copy(he_hbm.at[pl.ds(0, KC)], buf, gsem).wait()

        def _ss(ci, buf, ssem):
            pltpu.async_copy(buf, acc_sh.at[eib.at[ci, 0]], ssem, add=True)

        def _sw(ci, buf, ssem):
            pltpu.make_async_copy(he_hbm.at[pl.ds(0, KC)], buf, ssem).wait()

        _gs(0, rows_a, gsa)
        _gs(1, rows_b, gsb)
        _gw(0, rows_a, gsa)
        _process(0, eib, rows_a)
        _ss(0, rows_a, ssa)

        def _pbody(o, prefetch):
            _gw(o, rows_b, gsb)
            _process(o, eib, rows_b)
            _sw(o - 1, rows_a, ssa)
            _gs(o + 1, rows_a, gsa)
            _ss(o, rows_b, ssb)
            _gw(o + 1, rows_a, gsa)
            _process(o + 1, eib, rows_a)
            _sw(o, rows_b, ssb)
            if prefetch:
                _gs(o + 2, rows_b, gsb)
            _ss(o + 1, rows_a, ssa)

        def _pair(p, carry):
            _pbody(2 * p + 1, True)
            return carry
        lax.fori_loop(0, (CPB - 3) // 2, _pair, 0)
        _pbody(CPB - 2, False)
        _sw(CPB - 1, rows_a, ssa)

        if b + 2 < NB:
            _idx_start(b + 2, eib, ejb, isem)

    plsc.subcore_barrier()
    pltpu.sync_copy(acc_sh.at[pl.ds(base, RPS)], out_hbm.at[cid, pl.ds(base, RPS)])


@functools.cache
def _gat_sc():
    return pl.kernel(
        _gat_sc_body,
        out_type=jax.ShapeDtypeStruct((2, NP, HE_W), jnp.float32),
        mesh=_mesh(),
        compiler_params=pltpu.CompilerParams(needs_layout_passes=False, use_tc_tiling_on_sc=False),
        scratch_types=[
            pltpu.VMEM((CPB, 1, KC), jnp.int32),
            pltpu.VMEM((CPB, 1, KC), jnp.int32),
            pltpu.VMEM((CPB, 1, KC), jnp.int32),
            pltpu.VMEM((CPB, 1, KC), jnp.int32),
            pltpu.VMEM((NP,), jnp.float32),
            pltpu.VMEM((KC, HE_W), jnp.float32),
            pltpu.VMEM((KC, HE_W), jnp.float32),
            pltpu.VMEM((KC,), jnp.float32),
            pltpu.VMEM_SHARED((NP, HE_W), jnp.float32),
            pltpu.SemaphoreType.DMA,
            pltpu.SemaphoreType.DMA,
            pltpu.SemaphoreType.DMA,
            pltpu.SemaphoreType.DMA,
            pltpu.SemaphoreType.DMA,
            pltpu.SemaphoreType.DMA,
        ],
    )


def _take_sc_body(y_hbm, idx_hbm, out_hbm, idx_vm, rows_vm, sem):
    cid = lax.axis_index("c")
    sid = lax.axis_index("s")
    base = (cid * 16 + sid) * IPW
    pltpu.sync_copy(idx_hbm.at[pl.ds(base, IPW)], idx_vm)
    pltpu.async_copy(y_hbm.at[idx_vm], rows_vm, sem).wait()
    pltpu.sync_copy(rows_vm, out_hbm.at[pl.ds(base, IPW)])


@functools.cache
def _take_sc():
    return pl.kernel(
        _take_sc_body,
        out_type=jax.ShapeDtypeStruct((N_IDX, EMB), jnp.float32),
        mesh=_mesh(),
        compiler_params=pltpu.CompilerParams(needs_layout_passes=False, use_tc_tiling_on_sc=False),
        scratch_types=[
            pltpu.VMEM((IPW,), jnp.int32),
            pltpu.VMEM((IPW, EMB), jnp.float32),
            pltpu.SemaphoreType.DMA,
        ],
    )


def _a_pad(a1, a2):
    a = jnp.zeros((HID, 16), jnp.float32)
    return a.at[:, 0].set(a1).at[:, 1].set(a2)


def _bn_consts(gamma, beta, mean, var):
    sc = gamma / jnp.sqrt(var + 1e-5)
    sh = beta - mean * sc
    return sc.reshape(1, HID), sh.reshape(1, HID)


def kernel(features, edge_index, idx, W0, a1_0, a2_0, b0, gamma0, beta0,
           mean0, var0, W1, a1_1, a2_1, b1, gamma1, beta1, mean1, var1, Wd, bd):
    ei = edge_index[0].reshape(NWORK, NCH, 1, KC)
    ej = edge_index[1].reshape(NWORK, NCH, 1, KC)

    sc0, sh0 = _bn_consts(gamma0, beta0, mean0, var0)
    sc1, sh1 = _bn_consts(gamma1, beta1, mean1, var1)

    xp = jnp.pad(features, ((0, NP - N), (0, 0)))
    he0, sp0 = _encode(xp, W0, _a_pad(a1_0, a2_0))
    acc0 = _gat_sc()(he0, sp0.reshape(NP), ei, ej)
    he1, sp1 = _agg_encode(acc0, b0.reshape(1, HID), sc0, sh0, W1,
                           _a_pad(a1_1, a2_1))
    acc1 = _gat_sc()(he1, sp1.reshape(NP), ei, ej)
    y = _agg_dense(acc1, b1.reshape(1, HID), sc1, sh1, Wd, bd.reshape(1, EMB))
    return _take_sc()(y, idx)


# speedup vs baseline: 1.2409x; 1.2402x over previous
import functools

import jax
import jax.numpy as jnp
from jax import lax
from jax.experimental import pallas as pl
from jax.experimental.pallas import tpu as pltpu
from jax.experimental.pallas import tpu_sc as plsc

N = 10000
E = 320000
F_IN = 128
HID = 128
EMB = 64
N_IDX = 2048

HE_W = 136
NP = 10240
NWORK = 32
EPW = E // NWORK
KC = 80
NCH = EPW // KC
NB = 5
CPB = NCH // NB
RPS = NP // 16
IPW = N_IDX // NWORK

@functools.cache
def _mesh():
    return plsc.VectorSubcoreMesh(
        core_axis_name="c", subcore_axis_name="s", num_cores=2, num_subcores=16)


_BLK = 1024


def _enc_tail(h, s, he_ref, sp_ref):
    lane = lax.broadcasted_iota(jnp.int32, (_BLK, HE_W - HID), 1)
    extra = jnp.where(lane == 0, s[:, 1:2], 0.0)
    he_ref[...] = jnp.concatenate([h, extra], axis=1)
    sp_ref[...] = s[:, 0:1]


def _encode_body(x_ref, w_ref, a_ref, he_ref, sp_ref):
    h = jnp.dot(x_ref[...], w_ref[...], preferred_element_type=jnp.float32)
    s = jnp.dot(h, a_ref[...], preferred_element_type=jnp.float32)
    _enc_tail(h, s, he_ref, sp_ref)


def _agg(acc_ref, b_ref, sc_ref, sh_ref):
    a = acc_ref[0] + acc_ref[1]
    x = a[:, :HID] / jnp.maximum(a[:, HID:HID + 1], 1e-9) + b_ref[...]
    return jnp.maximum(x * sc_ref[...] + sh_ref[...], 0.0)


def _agg_encode_body(acc_ref, b_ref, sc_ref, sh_ref, w_ref, a_ref, he_ref, sp_ref):
    x = _agg(acc_ref, b_ref, sc_ref, sh_ref)
    h = jnp.dot(x, w_ref[...], preferred_element_type=jnp.float32)
    s = jnp.dot(h, a_ref[...], preferred_element_type=jnp.float32)
    _enc_tail(h, s, he_ref, sp_ref)


def _agg_dense_body(acc_ref, b_ref, sc_ref, sh_ref, wd_ref, bd_ref, y_ref):
    x = _agg(acc_ref, b_ref, sc_ref, sh_ref)
    y_ref[...] = (
        jnp.dot(x, wd_ref[...], preferred_element_type=jnp.float32) + bd_ref[...])


def _vec_spec():
    return pl.BlockSpec((1, HID), lambda i: (0, 0))


_encode = pl.pallas_call(
    _encode_body,
    grid=(NP // _BLK,),
    in_specs=[
        pl.BlockSpec((_BLK, F_IN), lambda i: (i, 0)),
        pl.BlockSpec((F_IN, HID), lambda i: (0, 0)),
        pl.BlockSpec((HID, 16), lambda i: (0, 0)),
    ],
    out_specs=[
        pl.BlockSpec((_BLK, HE_W), lambda i: (i, 0)),
        pl.BlockSpec((_BLK, 1), lambda i: (i, 0)),
    ],
    out_shape=[
        jax.ShapeDtypeStruct((NP, HE_W), jnp.float32),
        jax.ShapeDtypeStruct((NP, 1), jnp.float32),
    ],
)

_agg_encode = pl.pallas_call(
    _agg_encode_body,
    grid=(NP // _BLK,),
    in_specs=[
        pl.BlockSpec((2, _BLK, HE_W), lambda i: (0, i, 0)),
        _vec_spec(), _vec_spec(), _vec_spec(),
        pl.BlockSpec((HID, HID), lambda i: (0, 0)),
        pl.BlockSpec((HID, 16), lambda i: (0, 0)),
    ],
    out_specs=[
        pl.BlockSpec((_BLK, HE_W), lambda i: (i, 0)),
        pl.BlockSpec((_BLK, 1), lambda i: (i, 0)),
    ],
    out_shape=[
        jax.ShapeDtypeStruct((NP, HE_W), jnp.float32),
        jax.ShapeDtypeStruct((NP, 1), jnp.float32),
    ],
)

_agg_dense = pl.pallas_call(
    _agg_dense_body,
    grid=(NP // _BLK,),
    in_specs=[
        pl.BlockSpec((2, _BLK, HE_W), lambda i: (0, i, 0)),
        _vec_spec(), _vec_spec(), _vec_spec(),
        pl.BlockSpec((HID, EMB), lambda i: (0, 0)),
        pl.BlockSpec((1, EMB), lambda i: (0, 0)),
    ],
    out_specs=pl.BlockSpec((_BLK, EMB), lambda i: (i, 0)),
    out_shape=jax.ShapeDtypeStruct((NP, EMB), jnp.float32),
)


def _gat_sc_body(he_hbm, sp_hbm, ei_hbm, ej_hbm, out_hbm,
                 ei_a, ej_a, ei_b, ej_b, sp_vm, rows_a, rows_b, w_vm, acc_sh,
                 gsa, gsb, isa, isb):
    cid = lax.axis_index("c")
    sid = lax.axis_index("s")
    wid = cid * 16 + sid

    pltpu.sync_copy(sp_hbm, sp_vm)

    lane16 = lax.iota(jnp.int32, 16)

    def _zr(r, carry):
        z = jnp.zeros((16,), jnp.float32)
        for c in range(HID // 16):
            rows_a[r, pl.ds(c * 16, 16)] = z
        plsc.store_scatter(rows_a, [jnp.full((16,), r, jnp.int32), HID + lane16],
                           z, mask=lane16 < HE_W - HID)
        return carry
    lax.fori_loop(0, KC, _zr, 0)
    base = sid * RPS
    for k in range(RPS // KC):
        pltpu.sync_copy(rows_a, acc_sh.at[pl.ds(base + k * KC, KC)])
    plsc.subcore_barrier()

    def _process(ci, eib, buf):
        for g in range(KC // 16):
            eiv = eib[ci, 0, pl.ds(g * 16, 16)]
            spv = plsc.load_gather(sp_vm, [eiv])
            ridx = lane16 + g * 16
            sqv = plsc.load_gather(buf, [ridx, jnp.full((16,), HID, jnp.int32)])
            ev = spv + sqv
            ev = jnp.where(ev > 0, ev, ev * 0.2)
            w_vm[pl.ds(g * 16, 16)] = jnp.exp(ev)

        def _srow(r, c2):
            wv = plsc.load_gather(w_vm, [jnp.full((16,), r, jnp.int32)])
            for c in range(HID // 16):
                buf[r, pl.ds(c * 16, 16)] = buf[r, pl.ds(c * 16, 16)] * wv
            plsc.store_scatter(buf, [jnp.full((16,), r, jnp.int32), HID + lane16],
                               wv, mask=lane16 < HE_W - HID)
            return c2
        lax.fori_loop(0, KC, _srow, 0)

        pltpu.sync_copy(buf, acc_sh.at[eib.at[ci, 0]], add=True)

    idx_bufs = [(ei_a, ej_a, isa), (ei_b, ej_b, isb)]

    def _idx_start(b, eib, ejb, isem):
        pltpu.async_copy(ei_hbm.at[wid, pl.ds(b * CPB, CPB)], eib, isem)
        pltpu.async_copy(ej_hbm.at[wid, pl.ds(b * CPB, CPB)], ejb, isem)

    def _idx_wait(b, eib, ejb, isem):
        pltpu.make_async_copy(ei_hbm.at[wid, pl.ds(b * CPB, CPB)], eib, isem).wait()
        pltpu.make_async_copy(ej_hbm.at[wid, pl.ds(b * CPB, CPB)], ejb, isem).wait()

    _idx_start(0, ei_a, ej_a, isa)
    _idx_start(1, ei_b, ej_b, isb)

    for b in range(NB):
        eib, ejb, isem = idx_bufs[b % 2]
        _idx_wait(b, eib, ejb, isem)

        def _gs(ci, buf, gsem):
            pltpu.async_copy(he_hbm.at[ejb.at[ci, 0]], buf, gsem)

        def _gw(ci, buf, gsem):
            pltpu.make_async_copy(he_hbm.at[pl.ds(0, KC)], buf, gsem).wait()

        _gs(0, rows_a, gsa)

        def _pair(p, carry):
            a = 2 * p
            _gs(a + 1, rows_b, gsb)
            _gw(a, rows_a, gsa)
            _process(a, eib, rows_a)
            _gs(a + 2, rows_a, gsa)
            _gw(a + 1, rows_b, gsb)
            _process(a + 1, eib, rows_b)
            return carry
        lax.fori_loop(0, (CPB - 1) // 2, _pair, 0)
        _gw(CPB - 1, rows_a, gsa)
        _process(CPB - 1, eib, rows_a)

        if b + 2 < NB:
            _idx_start(b + 2, eib, ejb, isem)

    plsc.subcore_barrier()
    pltpu.sync_copy(acc_sh.at[pl.ds(base, RPS)], out_hbm.at[cid, pl.ds(base, RPS)])


@functools.cache
def _gat_sc():
    return pl.kernel(
        _gat_sc_body,
        out_type=jax.ShapeDtypeStruct((2, NP, HE_W), jnp.float32),
        mesh=_mesh(),
        compiler_params=pltpu.CompilerParams(needs_layout_passes=False, use_tc_tiling_on_sc=False),
        scratch_types=[
            pltpu.VMEM((CPB, 1, KC), jnp.int32),
            pltpu.VMEM((CPB, 1, KC), jnp.int32),
            pltpu.VMEM((CPB, 1, KC), jnp.int32),
            pltpu.VMEM((CPB, 1, KC), jnp.int32),
            pltpu.VMEM((NP,), jnp.float32),
            pltpu.VMEM((KC, HE_W), jnp.float32),
            pltpu.VMEM((KC, HE_W), jnp.float32),
            pltpu.VMEM((KC,), jnp.float32),
            pltpu.VMEM_SHARED((NP, HE_W), jnp.float32),
            pltpu.SemaphoreType.DMA,
            pltpu.SemaphoreType.DMA,
            pltpu.SemaphoreType.DMA,
            pltpu.SemaphoreType.DMA,
        ],
    )


def _take_sc_body(y_hbm, idx_hbm, out_hbm, idx_vm, rows_vm, sem):
    cid = lax.axis_index("c")
    sid = lax.axis_index("s")
    base = (cid * 16 + sid) * IPW
    pltpu.sync_copy(idx_hbm.at[pl.ds(base, IPW)], idx_vm)
    pltpu.async_copy(y_hbm.at[idx_vm], rows_vm, sem).wait()
    pltpu.sync_copy(rows_vm, out_hbm.at[pl.ds(base, IPW)])


@functools.cache
def _take_sc():
    return pl.kernel(
        _take_sc_body,
        out_type=jax.ShapeDtypeStruct((N_IDX, EMB), jnp.float32),
        mesh=_mesh(),
        compiler_params=pltpu.CompilerParams(needs_layout_passes=False, use_tc_tiling_on_sc=False),
        scratch_types=[
            pltpu.VMEM((IPW,), jnp.int32),
            pltpu.VMEM((IPW, EMB), jnp.float32),
            pltpu.SemaphoreType.DMA,
        ],
    )


def _a_pad(a1, a2):
    a = jnp.zeros((HID, 16), jnp.float32)
    return a.at[:, 0].set(a1).at[:, 1].set(a2)


def _bn_consts(gamma, beta, mean, var):
    sc = gamma / jnp.sqrt(var + 1e-5)
    sh = beta - mean * sc
    return sc.reshape(1, HID), sh.reshape(1, HID)


def kernel(features, edge_index, idx, W0, a1_0, a2_0, b0, gamma0, beta0,
           mean0, var0, W1, a1_1, a2_1, b1, gamma1, beta1, mean1, var1, Wd, bd):
    ei = edge_index[0].reshape(NWORK, NCH, 1, KC)
    ej = edge_index[1].reshape(NWORK, NCH, 1, KC)

    sc0, sh0 = _bn_consts(gamma0, beta0, mean0, var0)
    sc1, sh1 = _bn_consts(gamma1, beta1, mean1, var1)

    xp = jnp.pad(features, ((0, NP - N), (0, 0)))
    he0, sp0 = _encode(xp, W0, _a_pad(a1_0, a2_0))
    acc0 = _gat_sc()(he0, sp0.reshape(NP), ei, ej)
    he1, sp1 = _agg_encode(acc0, b0.reshape(1, HID), sc0, sh0, W1,
                           _a_pad(a1_1, a2_1))
    acc1 = _gat_sc()(he1, sp1.reshape(NP), ei, ej)
    y = _agg_dense(acc1, b1.reshape(1, HID), sc1, sh1, Wd, bd.reshape(1, EMB))
    return _take_sc()(y, idx)


# speedup vs baseline: 1.3346x; 1.0755x over previous
import functools

import jax
import jax.numpy as jnp
from jax import lax
from jax.experimental import pallas as pl
from jax.experimental.pallas import tpu as pltpu
from jax.experimental.pallas import tpu_sc as plsc

N = 10000
E = 320000
F_IN = 128
HID = 128
EMB = 64
N_IDX = 2048

HE_W = 136
NP = 10240
NWORK = 32
EPW = E // NWORK
KC = 80
NCH = EPW // KC
NB = 5
CPB = NCH // NB
RPS = NP // 16
IPW = N_IDX // NWORK

@functools.cache
def _mesh():
    return plsc.VectorSubcoreMesh(
        core_axis_name="c", subcore_axis_name="s", num_cores=2, num_subcores=16)


_BLK = 1024


def _enc_tail(h, s, he_ref, sp_ref):
    lane = lax.broadcasted_iota(jnp.int32, (_BLK, HE_W - HID), 1)
    extra = jnp.where(lane == 0, s[:, 1:2], 0.0)
    he_ref[...] = jnp.concatenate([h, extra], axis=1)
    sp_ref[...] = s[:, 0:1]


def _encode_body(x_ref, w_ref, a_ref, he_ref, sp_ref):
    h = jnp.dot(x_ref[...], w_ref[...], preferred_element_type=jnp.float32)
    s = jnp.dot(h, a_ref[...], preferred_element_type=jnp.float32)
    _enc_tail(h, s, he_ref, sp_ref)


def _agg(acc_ref, b_ref, sc_ref, sh_ref):
    a = acc_ref[0] + acc_ref[1]
    x = a[:, :HID] / jnp.maximum(a[:, HID:HID + 1], 1e-9) + b_ref[...]
    return jnp.maximum(x * sc_ref[...] + sh_ref[...], 0.0)


def _agg_encode_body(acc_ref, b_ref, sc_ref, sh_ref, w_ref, a_ref, he_ref, sp_ref):
    x = _agg(acc_ref, b_ref, sc_ref, sh_ref)
    h = jnp.dot(x, w_ref[...], preferred_element_type=jnp.float32)
    s = jnp.dot(h, a_ref[...], preferred_element_type=jnp.float32)
    _enc_tail(h, s, he_ref, sp_ref)


def _agg_dense_body(acc_ref, b_ref, sc_ref, sh_ref, wd_ref, bd_ref, y_ref):
    x = _agg(acc_ref, b_ref, sc_ref, sh_ref)
    y_ref[...] = (
        jnp.dot(x, wd_ref[...], preferred_element_type=jnp.float32) + bd_ref[...])


def _vec_spec():
    return pl.BlockSpec((1, HID), lambda i: (0, 0))


_encode = pl.pallas_call(
    _encode_body,
    grid=(NP // _BLK,),
    in_specs=[
        pl.BlockSpec((_BLK, F_IN), lambda i: (i, 0)),
        pl.BlockSpec((F_IN, HID), lambda i: (0, 0)),
        pl.BlockSpec((HID, 16), lambda i: (0, 0)),
    ],
    out_specs=[
        pl.BlockSpec((_BLK, HE_W), lambda i: (i, 0)),
        pl.BlockSpec((_BLK, 1), lambda i: (i, 0)),
    ],
    out_shape=[
        jax.ShapeDtypeStruct((NP, HE_W), jnp.float32),
        jax.ShapeDtypeStruct((NP, 1), jnp.float32),
    ],
)

_agg_encode = pl.pallas_call(
    _agg_encode_body,
    grid=(NP // _BLK,),
    in_specs=[
        pl.BlockSpec((2, _BLK, HE_W), lambda i: (0, i, 0)),
        _vec_spec(), _vec_spec(), _vec_spec(),
        pl.BlockSpec((HID, HID), lambda i: (0, 0)),
        pl.BlockSpec((HID, 16), lambda i: (0, 0)),
    ],
    out_specs=[
        pl.BlockSpec((_BLK, HE_W), lambda i: (i, 0)),
        pl.BlockSpec((_BLK, 1), lambda i: (i, 0)),
    ],
    out_shape=[
        jax.ShapeDtypeStruct((NP, HE_W), jnp.float32),
        jax.ShapeDtypeStruct((NP, 1), jnp.float32),
    ],
)

_agg_dense = pl.pallas_call(
    _agg_dense_body,
    grid=(NP // _BLK,),
    in_specs=[
        pl.BlockSpec((2, _BLK, HE_W), lambda i: (0, i, 0)),
        _vec_spec(), _vec_spec(), _vec_spec(),
        pl.BlockSpec((HID, EMB), lambda i: (0, 0)),
        pl.BlockSpec((1, EMB), lambda i: (0, 0)),
    ],
    out_specs=pl.BlockSpec((_BLK, EMB), lambda i: (i, 0)),
    out_shape=jax.ShapeDtypeStruct((NP, EMB), jnp.float32),
)


_final_dense = pl.pallas_call(
    _agg_dense_body,
    grid=(N_IDX // 1024,),
    in_specs=[
        pl.BlockSpec((2, 1024, HE_W), lambda i: (0, i, 0)),
        _vec_spec(), _vec_spec(), _vec_spec(),
        pl.BlockSpec((HID, EMB), lambda i: (0, 0)),
        pl.BlockSpec((1, EMB), lambda i: (0, 0)),
    ],
    out_specs=pl.BlockSpec((1024, EMB), lambda i: (i, 0)),
    out_shape=jax.ShapeDtypeStruct((N_IDX, EMB), jnp.float32),
)


def _gat_core(he_hbm, sp_hbm, ei_hbm, ej_hbm,
              ei_a, ej_a, ei_b, ej_b, sp_vm, rows_a, rows_b, w_vm, acc_sh,
              gsa, gsb, isa, isb):
    cid = lax.axis_index("c")
    sid = lax.axis_index("s")
    wid = cid * 16 + sid

    pltpu.sync_copy(sp_hbm, sp_vm)

    lane16 = lax.iota(jnp.int32, 16)

    def _zr(r, carry):
        z = jnp.zeros((16,), jnp.float32)
        for c in range(HID // 16):
            rows_a[r, pl.ds(c * 16, 16)] = z
        plsc.store_scatter(rows_a, [jnp.full((16,), r, jnp.int32), HID + lane16],
                           z, mask=lane16 < HE_W - HID)
        return carry
    lax.fori_loop(0, KC, _zr, 0)
    base = sid * RPS
    for k in range(RPS // KC):
        pltpu.sync_copy(rows_a, acc_sh.at[pl.ds(base + k * KC, KC)])
    plsc.subcore_barrier()

    def _process(ci, eib, buf):
        for g in range(KC // 16):
            eiv = eib[ci, 0, pl.ds(g * 16, 16)]
            spv = plsc.load_gather(sp_vm, [eiv])
            ridx = lane16 + g * 16
            sqv = plsc.load_gather(buf, [ridx, jnp.full((16,), HID, jnp.int32)])
            ev = spv + sqv
            ev = jnp.where(ev > 0, ev, ev * 0.2)
            w_vm[pl.ds(g * 16, 16)] = jnp.exp(ev)

        def _srow(r, c2):
            wv = plsc.load_gather(w_vm, [jnp.full((16,), r, jnp.int32)])
            for c in range(HID // 16):
                buf[r, pl.ds(c * 16, 16)] = buf[r, pl.ds(c * 16, 16)] * wv
            plsc.store_scatter(buf, [jnp.full((16,), r, jnp.int32), HID + lane16],
                               wv, mask=lane16 < HE_W - HID)
            return c2
        lax.fori_loop(0, KC, _srow, 0)

        pltpu.sync_copy(buf, acc_sh.at[eib.at[ci, 0]], add=True)

    idx_bufs = [(ei_a, ej_a, isa), (ei_b, ej_b, isb)]

    def _idx_start(b, eib, ejb, isem):
        pltpu.async_copy(ei_hbm.at[wid, pl.ds(b * CPB, CPB)], eib, isem)
        pltpu.async_copy(ej_hbm.at[wid, pl.ds(b * CPB, CPB)], ejb, isem)

    def _idx_wait(b, eib, ejb, isem):
        pltpu.make_async_copy(ei_hbm.at[wid, pl.ds(b * CPB, CPB)], eib, isem).wait()
        pltpu.make_async_copy(ej_hbm.at[wid, pl.ds(b * CPB, CPB)], ejb, isem).wait()

    _idx_start(0, ei_a, ej_a, isa)
    _idx_start(1, ei_b, ej_b, isb)

    for b in range(NB):
        eib, ejb, isem = idx_bufs[b % 2]
        _idx_wait(b, eib, ejb, isem)

        def _gs(ci, buf, gsem):
            pltpu.async_copy(he_hbm.at[ejb.at[ci, 0]], buf, gsem)

        def _gw(ci, buf, gsem):
            pltpu.make_async_copy(he_hbm.at[pl.ds(0, KC)], buf, gsem).wait()

        _gs(0, rows_a, gsa)

        def _pair(p, carry):
            a = 2 * p
            _gs(a + 1, rows_b, gsb)
            _gw(a, rows_a, gsa)
            _process(a, eib, rows_a)
            _gs(a + 2, rows_a, gsa)
            _gw(a + 1, rows_b, gsb)
            _process(a + 1, eib, rows_b)
            return carry
        lax.fori_loop(0, (CPB - 1) // 2, _pair, 0)
        _gw(CPB - 1, rows_a, gsa)
        _process(CPB - 1, eib, rows_a)

        if b + 2 < NB:
            _idx_start(b + 2, eib, ejb, isem)

    plsc.subcore_barrier()
    return cid, sid, base


def _gat_sc_body(he_hbm, sp_hbm, ei_hbm, ej_hbm, out_hbm,
                 ei_a, ej_a, ei_b, ej_b, sp_vm, rows_a, rows_b, w_vm, acc_sh,
                 gsa, gsb, isa, isb):
    cid, sid, base = _gat_core(he_hbm, sp_hbm, ei_hbm, ej_hbm,
                               ei_a, ej_a, ei_b, ej_b, sp_vm, rows_a, rows_b,
                               w_vm, acc_sh, gsa, gsb, isa, isb)
    pltpu.sync_copy(acc_sh.at[pl.ds(base, RPS)], out_hbm.at[cid, pl.ds(base, RPS)])


def _gat_sel_body(he_hbm, sp_hbm, ei_hbm, ej_hbm, idx_hbm, sel_hbm,
                  ei_a, ej_a, ei_b, ej_b, sp_vm, rows_a, rows_b, w_vm, acc_sh,
                  idx_vm, gsa, gsb, isa, isb):
    cid, sid, base = _gat_core(he_hbm, sp_hbm, ei_hbm, ej_hbm,
                               ei_a, ej_a, ei_b, ej_b, sp_vm, rows_a, rows_b,
                               w_vm, acc_sh, gsa, gsb, isa, isb)
    for h2 in range(N_IDX // (16 * IPW)):
        b2 = sid * (N_IDX // 16) + h2 * IPW
        pltpu.sync_copy(idx_hbm.at[pl.ds(b2, IPW)], idx_vm)
        pltpu.async_copy(acc_sh.at[idx_vm], rows_a.at[pl.ds(0, IPW)], gsa).wait()
        pltpu.sync_copy(rows_a.at[pl.ds(0, IPW)], sel_hbm.at[cid, pl.ds(b2, IPW)])


@functools.cache
def _gat_sc():
    return pl.kernel(
        _gat_sc_body,
        out_type=jax.ShapeDtypeStruct((2, NP, HE_W), jnp.float32),
        mesh=_mesh(),
        compiler_params=pltpu.CompilerParams(needs_layout_passes=False, use_tc_tiling_on_sc=False),
        scratch_types=[
            pltpu.VMEM((CPB, 1, KC), jnp.int32),
            pltpu.VMEM((CPB, 1, KC), jnp.int32),
            pltpu.VMEM((CPB, 1, KC), jnp.int32),
            pltpu.VMEM((CPB, 1, KC), jnp.int32),
            pltpu.VMEM((NP,), jnp.float32),
            pltpu.VMEM((KC, HE_W), jnp.float32),
            pltpu.VMEM((KC, HE_W), jnp.float32),
            pltpu.VMEM((KC,), jnp.float32),
            pltpu.VMEM_SHARED((NP, HE_W), jnp.float32),
            pltpu.SemaphoreType.DMA,
            pltpu.SemaphoreType.DMA,
            pltpu.SemaphoreType.DMA,
            pltpu.SemaphoreType.DMA,
        ],
    )


def _take_sc_body(y_hbm, idx_hbm, out_hbm, idx_vm, rows_vm, sem):
    cid = lax.axis_index("c")
    sid = lax.axis_index("s")
    base = (cid * 16 + sid) * IPW
    pltpu.sync_copy(idx_hbm.at[pl.ds(base, IPW)], idx_vm)
    pltpu.async_copy(y_hbm.at[idx_vm], rows_vm, sem).wait()
    pltpu.sync_copy(rows_vm, out_hbm.at[pl.ds(base, IPW)])


@functools.cache
def _gat_sel():
    return pl.kernel(
        _gat_sel_body,
        out_type=jax.ShapeDtypeStruct((2, N_IDX, HE_W), jnp.float32),
        mesh=_mesh(),
        compiler_params=pltpu.CompilerParams(needs_layout_passes=False, use_tc_tiling_on_sc=False),
        scratch_types=[
            pltpu.VMEM((CPB, 1, KC), jnp.int32),
            pltpu.VMEM((CPB, 1, KC), jnp.int32),
            pltpu.VMEM((CPB, 1, KC), jnp.int32),
            pltpu.VMEM((CPB, 1, KC), jnp.int32),
            pltpu.VMEM((NP,), jnp.float32),
            pltpu.VMEM((KC, HE_W), jnp.float32),
            pltpu.VMEM((KC, HE_W), jnp.float32),
            pltpu.VMEM((KC,), jnp.float32),
            pltpu.VMEM_SHARED((NP, HE_W), jnp.float32),
            pltpu.VMEM((IPW,), jnp.int32),
            pltpu.SemaphoreType.DMA,
            pltpu.SemaphoreType.DMA,
            pltpu.SemaphoreType.DMA,
            pltpu.SemaphoreType.DMA,
        ],
    )


@functools.cache
def _take_sc():
    return pl.kernel(
        _take_sc_body,
        out_type=jax.ShapeDtypeStruct((N_IDX, EMB), jnp.float32),
        mesh=_mesh(),
        compiler_params=pltpu.CompilerParams(needs_layout_passes=False, use_tc_tiling_on_sc=False),
        scratch_types=[
            pltpu.VMEM((IPW,), jnp.int32),
            pltpu.VMEM((IPW, EMB), jnp.float32),
            pltpu.SemaphoreType.DMA,
        ],
    )


def _a_pad(a1, a2):
    a = jnp.zeros((HID, 16), jnp.float32)
    return a.at[:, 0].set(a1).at[:, 1].set(a2)


def _bn_consts(gamma, beta, mean, var):
    sc = gamma / jnp.sqrt(var + 1e-5)
    sh = beta - mean * sc
    return sc.reshape(1, HID), sh.reshape(1, HID)


def kernel(features, edge_index, idx, W0, a1_0, a2_0, b0, gamma0, beta0,
           mean0, var0, W1, a1_1, a2_1, b1, gamma1, beta1, mean1, var1, Wd, bd):
    ei = edge_index[0].reshape(NWORK, NCH, 1, KC)
    ej = edge_index[1].reshape(NWORK, NCH, 1, KC)

    sc0, sh0 = _bn_consts(gamma0, beta0, mean0, var0)
    sc1, sh1 = _bn_consts(gamma1, beta1, mean1, var1)

    xp = jnp.pad(features, ((0, NP - N), (0, 0)))
    he0, sp0 = _encode(xp, W0, _a_pad(a1_0, a2_0))
    acc0 = _gat_sc()(he0, sp0.reshape(NP), ei, ej)
    he1, sp1 = _agg_encode(acc0, b0.reshape(1, HID), sc0, sh0, W1,
                           _a_pad(a1_1, a2_1))
    sel = _gat_sel()(he1, sp1.reshape(NP), ei, ej, idx)
    return _final_dense(sel, b1.reshape(1, HID), sc1, sh1, Wd, bd.reshape(1, EMB))


# speedup vs baseline: 1.6880x; 1.2648x over previous
import functools

import jax
import jax.numpy as jnp
from jax import lax
from jax.experimental import pallas as pl
from jax.experimental.pallas import tpu as pltpu
from jax.experimental.pallas import tpu_sc as plsc

N = 10000
E = 320000
F_IN = 128
HID = 128
EMB = 64
N_IDX = 2048

HE_W = 136
NP = 10240
NWORK = 32
EPW = E // NWORK
KC = 80
NCH = EPW // KC
NB = 5
CPB = NCH // NB
RPS = NP // 16
IPW = N_IDX // NWORK

@functools.cache
def _mesh():
    return plsc.VectorSubcoreMesh(
        core_axis_name="c", subcore_axis_name="s", num_cores=2, num_subcores=16)


_BLK = 1024


def _enc_tail(h, s, he_ref, sp_ref):
    lane = lax.broadcasted_iota(jnp.int32, (_BLK, HE_W - HID), 1)
    extra = jnp.where(lane == 0, s[:, 1:2], 0.0)
    he_ref[...] = jnp.concatenate([h, extra], axis=1)
    sp_ref[...] = s[:, 0:1]


def _encode_body(x_ref, w_ref, a_ref, he_ref, sp_ref):
    h = jnp.dot(x_ref[...], w_ref[...], preferred_element_type=jnp.float32)
    s = jnp.dot(h, a_ref[...], preferred_element_type=jnp.float32)
    _enc_tail(h, s, he_ref, sp_ref)


def _agg(acc_ref, b_ref, sc_ref, sh_ref):
    a = acc_ref[0] + acc_ref[1]
    x = a[:, :HID] / jnp.maximum(a[:, HID:HID + 1], 1e-9) + b_ref[...]
    return jnp.maximum(x * sc_ref[...] + sh_ref[...], 0.0)


def _agg_encode_body(acc_ref, b_ref, sc_ref, sh_ref, w_ref, a_ref, he_ref, sp_ref):
    x = _agg(acc_ref, b_ref, sc_ref, sh_ref)
    h = jnp.dot(x, w_ref[...], preferred_element_type=jnp.float32)
    s = jnp.dot(h, a_ref[...], preferred_element_type=jnp.float32)
    _enc_tail(h, s, he_ref, sp_ref)


def _agg_dense_body(acc_ref, b_ref, sc_ref, sh_ref, wd_ref, bd_ref, y_ref):
    x = _agg(acc_ref, b_ref, sc_ref, sh_ref)
    y_ref[...] = (
        jnp.dot(x, wd_ref[...], preferred_element_type=jnp.float32) + bd_ref[...])


def _vec_spec():
    return pl.BlockSpec((1, HID), lambda i: (0, 0))


_encode = pl.pallas_call(
    _encode_body,
    grid=(NP // _BLK,),
    in_specs=[
        pl.BlockSpec((_BLK, F_IN), lambda i: (i, 0)),
        pl.BlockSpec((F_IN, HID), lambda i: (0, 0)),
        pl.BlockSpec((HID, 16), lambda i: (0, 0)),
    ],
    out_specs=[
        pl.BlockSpec((_BLK, HE_W), lambda i: (i, 0)),
        pl.BlockSpec((_BLK, 1), lambda i: (i, 0)),
    ],
    out_shape=[
        jax.ShapeDtypeStruct((NP, HE_W), jnp.float32),
        jax.ShapeDtypeStruct((NP, 1), jnp.float32),
    ],
)

_agg_encode = pl.pallas_call(
    _agg_encode_body,
    grid=(NP // _BLK,),
    in_specs=[
        pl.BlockSpec((2, _BLK, HE_W), lambda i: (0, i, 0)),
        _vec_spec(), _vec_spec(), _vec_spec(),
        pl.BlockSpec((HID, HID), lambda i: (0, 0)),
        pl.BlockSpec((HID, 16), lambda i: (0, 0)),
    ],
    out_specs=[
        pl.BlockSpec((_BLK, HE_W), lambda i: (i, 0)),
        pl.BlockSpec((_BLK, 1), lambda i: (i, 0)),
    ],
    out_shape=[
        jax.ShapeDtypeStruct((NP, HE_W), jnp.float32),
        jax.ShapeDtypeStruct((NP, 1), jnp.float32),
    ],
)

_agg_dense = pl.pallas_call(
    _agg_dense_body,
    grid=(NP // _BLK,),
    in_specs=[
        pl.BlockSpec((2, _BLK, HE_W), lambda i: (0, i, 0)),
        _vec_spec(), _vec_spec(), _vec_spec(),
        pl.BlockSpec((HID, EMB), lambda i: (0, 0)),
        pl.BlockSpec((1, EMB), lambda i: (0, 0)),
    ],
    out_specs=pl.BlockSpec((_BLK, EMB), lambda i: (i, 0)),
    out_shape=jax.ShapeDtypeStruct((NP, EMB), jnp.float32),
)


_final_dense = pl.pallas_call(
    _agg_dense_body,
    grid=(N_IDX // 1024,),
    in_specs=[
        pl.BlockSpec((2, 1024, HE_W), lambda i: (0, i, 0)),
        _vec_spec(), _vec_spec(), _vec_spec(),
        pl.BlockSpec((HID, EMB), lambda i: (0, 0)),
        pl.BlockSpec((1, EMB), lambda i: (0, 0)),
    ],
    out_specs=pl.BlockSpec((1024, EMB), lambda i: (i, 0)),
    out_shape=jax.ShapeDtypeStruct((N_IDX, EMB), jnp.float32),
)


def _gat_core(he_hbm, sp_hbm, ei_hbm, ej_hbm,
              ei_a, ej_a, ei_b, ej_b, sp_vm, rows_a, rows_b, w_vm, acc_sh,
              gsa, gsb, isa, isb):
    cid = lax.axis_index("c")
    sid = lax.axis_index("s")
    wid = cid * 16 + sid

    pltpu.sync_copy(sp_hbm, sp_vm)

    lane16 = lax.iota(jnp.int32, 16)

    def _zr(r, carry):
        z = jnp.zeros((16,), jnp.float32)
        for c in range(HID // 16):
            rows_a[r, pl.ds(c * 16, 16)] = z
        plsc.store_scatter(rows_a, [jnp.full((16,), r, jnp.int32), HID + lane16],
                           z, mask=lane16 < HE_W - HID)
        return carry
    lax.fori_loop(0, KC, _zr, 0)
    base = sid * RPS
    for k in range(RPS // KC):
        pltpu.sync_copy(rows_a, acc_sh.at[pl.ds(base + k * KC, KC)])
    plsc.subcore_barrier()

    def _process(ci, eib, buf):
        for g in range(KC // 16):
            eiv = eib[ci, 0, pl.ds(g * 16, 16)]
            spv = plsc.load_gather(sp_vm, [eiv])
            ridx = lane16 + g * 16
            sqv = plsc.load_gather(buf, [ridx, jnp.full((16,), HID, jnp.int32)])
            ev = spv + sqv
            ev = jnp.where(ev > 0, ev, ev * 0.2)
            w_vm[pl.ds(g * 16, 16)] = jnp.exp(ev)

        @plsc.parallel_loop(0, KC, unroll=2)
        def _srow(r):
            wv = plsc.load_gather(w_vm, [jnp.full((16,), r, jnp.int32)])
            for c in range(HID // 16):
                buf[r, pl.ds(c * 16, 16)] = buf[r, pl.ds(c * 16, 16)] * wv
            plsc.store_scatter(buf, [jnp.full((16,), r, jnp.int32), HID + lane16],
                               wv, mask=lane16 < HE_W - HID)

        pltpu.sync_copy(buf, acc_sh.at[eib.at[ci, 0]], add=True)

    idx_bufs = [(ei_a, ej_a, isa), (ei_b, ej_b, isb)]

    def _idx_start(b, eib, ejb, isem):
        pltpu.async_copy(ei_hbm.at[wid, pl.ds(b * CPB, CPB)], eib, isem)
        pltpu.async_copy(ej_hbm.at[wid, pl.ds(b * CPB, CPB)], ejb, isem)

    def _idx_wait(b, eib, ejb, isem):
        pltpu.make_async_copy(ei_hbm.at[wid, pl.ds(b * CPB, CPB)], eib, isem).wait()
        pltpu.make_async_copy(ej_hbm.at[wid, pl.ds(b * CPB, CPB)], ejb, isem).wait()

    _idx_start(0, ei_a, ej_a, isa)
    _idx_start(1, ei_b, ej_b, isb)

    for b in range(NB):
        eib, ejb, isem = idx_bufs[b % 2]
        _idx_wait(b, eib, ejb, isem)

        def _gs(ci, buf, gsem):
            pltpu.async_copy(he_hbm.at[ejb.at[ci, 0]], buf, gsem)

        def _gw(ci, buf, gsem):
            pltpu.make_async_copy(he_hbm.at[pl.ds(0, KC)], buf, gsem).wait()

        _gs(0, rows_a, gsa)

        def _pair(p, carry):
            a = 2 * p
            _gs(a + 1, rows_b, gsb)
            _gw(a, rows_a, gsa)
            _process(a, eib, rows_a)
            _gs(a + 2, rows_a, gsa)
            _gw(a + 1, rows_b, gsb)
            _process(a + 1, eib, rows_b)
            return carry
        lax.fori_loop(0, (CPB - 1) // 2, _pair, 0)
        _gw(CPB - 1, rows_a, gsa)
        _process(CPB - 1, eib, rows_a)

        if b + 2 < NB:
            _idx_start(b + 2, eib, ejb, isem)

    plsc.subcore_barrier()
    return cid, sid, base


def _gat_sc_body(he_hbm, sp_hbm, ei_hbm, ej_hbm, out_hbm,
                 ei_a, ej_a, ei_b, ej_b, sp_vm, rows_a, rows_b, w_vm, acc_sh,
                 gsa, gsb, isa, isb):
    cid, sid, base = _gat_core(he_hbm, sp_hbm, ei_hbm, ej_hbm,
                               ei_a, ej_a, ei_b, ej_b, sp_vm, rows_a, rows_b,
                               w_vm, acc_sh, gsa, gsb, isa, isb)
    pltpu.sync_copy(acc_sh.at[pl.ds(base, RPS)], out_hbm.at[cid, pl.ds(base, RPS)])


def _gat_sel_body(he_hbm, sp_hbm, ei_hbm, ej_hbm, idx_hbm, sel_hbm,
                  ei_a, ej_a, ei_b, ej_b, sp_vm, rows_a, rows_b, w_vm, acc_sh,
                  idx_vm, gsa, gsb, isa, isb):
    cid, sid, base = _gat_core(he_hbm, sp_hbm, ei_hbm, ej_hbm,
                               ei_a, ej_a, ei_b, ej_b, sp_vm, rows_a, rows_b,
                               w_vm, acc_sh, gsa, gsb, isa, isb)
    for h2 in range(N_IDX // (16 * IPW)):
        b2 = sid * (N_IDX // 16) + h2 * IPW
        pltpu.sync_copy(idx_hbm.at[pl.ds(b2, IPW)], idx_vm)
        pltpu.async_copy(acc_sh.at[idx_vm], rows_a.at[pl.ds(0, IPW)], gsa).wait()
        pltpu.sync_copy(rows_a.at[pl.ds(0, IPW)], sel_hbm.at[cid, pl.ds(b2, IPW)])


@functools.cache
def _gat_sc():
    return pl.kernel(
        _gat_sc_body,
        out_type=jax.ShapeDtypeStruct((2, NP, HE_W), jnp.float32),
        mesh=_mesh(),
        compiler_params=pltpu.CompilerParams(needs_layout_passes=False, use_tc_tiling_on_sc=False),
        scratch_types=[
            pltpu.VMEM((CPB, 1, KC), jnp.int32),
            pltpu.VMEM((CPB, 1, KC), jnp.int32),
            pltpu.VMEM((CPB, 1, KC), jnp.int32),
            pltpu.VMEM((CPB, 1, KC), jnp.int32),
            pltpu.VMEM((NP,), jnp.float32),
            pltpu.VMEM((KC, HE_W), jnp.float32),
            pltpu.VMEM((KC, HE_W), jnp.float32),
            pltpu.VMEM((KC,), jnp.float32),
            pltpu.VMEM_SHARED((NP, HE_W), jnp.float32),
            pltpu.SemaphoreType.DMA,
            pltpu.SemaphoreType.DMA,
            pltpu.SemaphoreType.DMA,
            pltpu.SemaphoreType.DMA,
        ],
    )


def _take_sc_body(y_hbm, idx_hbm, out_hbm, idx_vm, rows_vm, sem):
    cid = lax.axis_index("c")
    sid = lax.axis_index("s")
    base = (cid * 16 + sid) * IPW
    pltpu.sync_copy(idx_hbm.at[pl.ds(base, IPW)], idx_vm)
    pltpu.async_copy(y_hbm.at[idx_vm], rows_vm, sem).wait()
    pltpu.sync_copy(rows_vm, out_hbm.at[pl.ds(base, IPW)])


@functools.cache
def _gat_sel():
    return pl.kernel(
        _gat_sel_body,
        out_type=jax.ShapeDtypeStruct((2, N_IDX, HE_W), jnp.float32),
        mesh=_mesh(),
        compiler_params=pltpu.CompilerParams(needs_layout_passes=False, use_tc_tiling_on_sc=False),
        scratch_types=[
            pltpu.VMEM((CPB, 1, KC), jnp.int32),
            pltpu.VMEM((CPB, 1, KC), jnp.int32),
            pltpu.VMEM((CPB, 1, KC), jnp.int32),
            pltpu.VMEM((CPB, 1, KC), jnp.int32),
            pltpu.VMEM((NP,), jnp.float32),
            pltpu.VMEM((KC, HE_W), jnp.float32),
            pltpu.VMEM((KC, HE_W), jnp.float32),
            pltpu.VMEM((KC,), jnp.float32),
            pltpu.VMEM_SHARED((NP, HE_W), jnp.float32),
            pltpu.VMEM((IPW,), jnp.int32),
            pltpu.SemaphoreType.DMA,
            pltpu.SemaphoreType.DMA,
            pltpu.SemaphoreType.DMA,
            pltpu.SemaphoreType.DMA,
        ],
    )


@functools.cache
def _take_sc():
    return pl.kernel(
        _take_sc_body,
        out_type=jax.ShapeDtypeStruct((N_IDX, EMB), jnp.float32),
        mesh=_mesh(),
        compiler_params=pltpu.CompilerParams(needs_layout_passes=False, use_tc_tiling_on_sc=False),
        scratch_types=[
            pltpu.VMEM((IPW,), jnp.int32),
            pltpu.VMEM((IPW, EMB), jnp.float32),
            pltpu.SemaphoreType.DMA,
        ],
    )


def _a_pad(a1, a2):
    a = jnp.zeros((HID, 16), jnp.float32)
    return a.at[:, 0].set(a1).at[:, 1].set(a2)


def _bn_consts(gamma, beta, mean, var):
    sc = gamma / jnp.sqrt(var + 1e-5)
    sh = beta - mean * sc
    return sc.reshape(1, HID), sh.reshape(1, HID)


def kernel(features, edge_index, idx, W0, a1_0, a2_0, b0, gamma0, beta0,
           mean0, var0, W1, a1_1, a2_1, b1, gamma1, beta1, mean1, var1, Wd, bd):
    ei = edge_index[0].reshape(NWORK, NCH, 1, KC)
    ej = edge_index[1].reshape(NWORK, NCH, 1, KC)

    sc0, sh0 = _bn_consts(gamma0, beta0, mean0, var0)
    sc1, sh1 = _bn_consts(gamma1, beta1, mean1, var1)

    xp = jnp.pad(features, ((0, NP - N), (0, 0)))
    he0, sp0 = _encode(xp, W0, _a_pad(a1_0, a2_0))
    acc0 = _gat_sc()(he0, sp0.reshape(NP), ei, ej)
    he1, sp1 = _agg_encode(acc0, b0.reshape(1, HID), sc0, sh0, W1,
                           _a_pad(a1_1, a2_1))
    sel = _gat_sel()(he1, sp1.reshape(NP), ei, ej, idx)
    return _final_dense(sel, b1.reshape(1, HID), sc1, sh1, Wd, bd.reshape(1, EMB))


# speedup vs baseline: 1.6959x; 1.0047x over previous
import functools

import jax
import jax.numpy as jnp
from jax import lax
from jax.experimental import pallas as pl
from jax.experimental.pallas import tpu as pltpu
from jax.experimental.pallas import tpu_sc as plsc

N = 10000
E = 320000
F_IN = 128
HID = 128
EMB = 64
N_IDX = 2048

HE_W = 136
NP = 10240
NWORK = 32
EPW = E // NWORK
KC = 80
NCH = EPW // KC
NB = 5
CPB = NCH // NB
RPS = NP // 16
IPW = N_IDX // NWORK

@functools.cache
def _mesh():
    return plsc.VectorSubcoreMesh(
        core_axis_name="c", subcore_axis_name="s", num_cores=2, num_subcores=16)


_BLK = 1024


def _enc_tail(h, s, he_ref, sp_ref):
    lane = lax.broadcasted_iota(jnp.int32, (_BLK, HE_W - HID), 1)
    extra = jnp.where(lane == 0, s[:, 1:2], 0.0)
    he_ref[...] = jnp.concatenate([h, extra], axis=1)
    sp_ref[...] = s[:, 0:1]


def _encode_body(x_ref, w_ref, a_ref, he_ref, sp_ref):
    h = jnp.dot(x_ref[...], w_ref[...], preferred_element_type=jnp.float32)
    s = jnp.dot(h, a_ref[...], preferred_element_type=jnp.float32)
    _enc_tail(h, s, he_ref, sp_ref)


def _agg(acc_ref, b_ref, sc_ref, sh_ref):
    a = acc_ref[0] + acc_ref[1]
    x = a[:, :HID] / jnp.maximum(a[:, HID:HID + 1], 1e-9) + b_ref[...]
    return jnp.maximum(x * sc_ref[...] + sh_ref[...], 0.0)


def _agg_encode_body(acc_ref, b_ref, sc_ref, sh_ref, w_ref, a_ref, he_ref, sp_ref):
    x = _agg(acc_ref, b_ref, sc_ref, sh_ref)
    h = jnp.dot(x, w_ref[...], preferred_element_type=jnp.float32)
    s = jnp.dot(h, a_ref[...], preferred_element_type=jnp.float32)
    _enc_tail(h, s, he_ref, sp_ref)


def _agg_dense_body(acc_ref, b_ref, sc_ref, sh_ref, wd_ref, bd_ref, y_ref):
    x = _agg(acc_ref, b_ref, sc_ref, sh_ref)
    y_ref[...] = (
        jnp.dot(x, wd_ref[...], preferred_element_type=jnp.float32) + bd_ref[...])


def _vec_spec():
    return pl.BlockSpec((1, HID), lambda i: (0, 0))


_encode = pl.pallas_call(
    _encode_body,
    grid=(NP // _BLK,),
    in_specs=[
        pl.BlockSpec((_BLK, F_IN), lambda i: (i, 0)),
        pl.BlockSpec((F_IN, HID), lambda i: (0, 0)),
        pl.BlockSpec((HID, 16), lambda i: (0, 0)),
    ],
    out_specs=[
        pl.BlockSpec((_BLK, HE_W), lambda i: (i, 0)),
        pl.BlockSpec((_BLK, 1), lambda i: (i, 0)),
    ],
    out_shape=[
        jax.ShapeDtypeStruct((NP, HE_W), jnp.float32),
        jax.ShapeDtypeStruct((NP, 1), jnp.float32),
    ],
)

_agg_encode = pl.pallas_call(
    _agg_encode_body,
    grid=(NP // _BLK,),
    in_specs=[
        pl.BlockSpec((2, _BLK, HE_W), lambda i: (0, i, 0)),
        _vec_spec(), _vec_spec(), _vec_spec(),
        pl.BlockSpec((HID, HID), lambda i: (0, 0)),
        pl.BlockSpec((HID, 16), lambda i: (0, 0)),
    ],
    out_specs=[
        pl.BlockSpec((_BLK, HE_W), lambda i: (i, 0)),
        pl.BlockSpec((_BLK, 1), lambda i: (i, 0)),
    ],
    out_shape=[
        jax.ShapeDtypeStruct((NP, HE_W), jnp.float32),
        jax.ShapeDtypeStruct((NP, 1), jnp.float32),
    ],
)

_agg_dense = pl.pallas_call(
    _agg_dense_body,
    grid=(NP // _BLK,),
    in_specs=[
        pl.BlockSpec((2, _BLK, HE_W), lambda i: (0, i, 0)),
        _vec_spec(), _vec_spec(), _vec_spec(),
        pl.BlockSpec((HID, EMB), lambda i: (0, 0)),
        pl.BlockSpec((1, EMB), lambda i: (0, 0)),
    ],
    out_specs=pl.BlockSpec((_BLK, EMB), lambda i: (i, 0)),
    out_shape=jax.ShapeDtypeStruct((NP, EMB), jnp.float32),
)


_final_dense = pl.pallas_call(
    _agg_dense_body,
    grid=(N_IDX // 1024,),
    in_specs=[
        pl.BlockSpec((2, 1024, HE_W), lambda i: (0, i, 0)),
        _vec_spec(), _vec_spec(), _vec_spec(),
        pl.BlockSpec((HID, EMB), lambda i: (0, 0)),
        pl.BlockSpec((1, EMB), lambda i: (0, 0)),
    ],
    out_specs=pl.BlockSpec((1024, EMB), lambda i: (i, 0)),
    out_shape=jax.ShapeDtypeStruct((N_IDX, EMB), jnp.float32),
)


def _gat_core(he_hbm, sp_hbm, ei_hbm, ej_hbm,
              ei_a, ej_a, ei_b, ej_b, sp_vm, rows_a, rows_b, w_vm, acc_sh,
              gsa, gsb, isa, isb):
    cid = lax.axis_index("c")
    sid = lax.axis_index("s")
    wid = cid * 16 + sid

    pltpu.sync_copy(sp_hbm, sp_vm)

    lane16 = lax.iota(jnp.int32, 16)

    def _zr(r, carry):
        z = jnp.zeros((16,), jnp.float32)
        for c in range(HID // 16):
            rows_a[r, pl.ds(c * 16, 16)] = z
        plsc.store_scatter(rows_a, [jnp.full((16,), r, jnp.int32), HID + lane16],
                           z, mask=lane16 < HE_W - HID)
        return carry
    lax.fori_loop(0, KC, _zr, 0)
    base = sid * RPS
    for k in range(RPS // KC):
        pltpu.sync_copy(rows_a, acc_sh.at[pl.ds(base + k * KC, KC)])
    plsc.subcore_barrier()

    def _process(ci, eib, buf):
        @plsc.parallel_loop(0, KC // 16, unroll=2)
        def _wg(g):
            eiv = eib[ci, 0, pl.ds(g * 16, 16)]
            spv = plsc.load_gather(sp_vm, [eiv])
            ridx = lane16 + g * 16
            sqv = plsc.load_gather(buf, [ridx, jnp.full((16,), HID, jnp.int32)])
            ev = spv + sqv
            ev = jnp.where(ev > 0, ev, ev * 0.2)
            w_vm[pl.ds(g * 16, 16)] = jnp.exp(ev)

        @plsc.parallel_loop(0, KC, unroll=4)
        def _srow(r):
            wv = plsc.load_gather(w_vm, [jnp.full((16,), r, jnp.int32)])
            for c in range(HID // 16):
                buf[r, pl.ds(c * 16, 16)] = buf[r, pl.ds(c * 16, 16)] * wv
            plsc.store_scatter(buf, [jnp.full((16,), r, jnp.int32), HID + lane16],
                               wv, mask=lane16 < HE_W - HID)

        pltpu.sync_copy(buf, acc_sh.at[eib.at[ci, 0]], add=True)

    idx_bufs = [(ei_a, ej_a, isa), (ei_b, ej_b, isb)]

    def _idx_start(b, eib, ejb, isem):
        pltpu.async_copy(ei_hbm.at[wid, pl.ds(b * CPB, CPB)], eib, isem)
        pltpu.async_copy(ej_hbm.at[wid, pl.ds(b * CPB, CPB)], ejb, isem)

    def _idx_wait(b, eib, ejb, isem):
        pltpu.make_async_copy(ei_hbm.at[wid, pl.ds(b * CPB, CPB)], eib, isem).wait()
        pltpu.make_async_copy(ej_hbm.at[wid, pl.ds(b * CPB, CPB)], ejb, isem).wait()

    _idx_start(0, ei_a, ej_a, isa)
    _idx_start(1, ei_b, ej_b, isb)

    for b in range(NB):
        eib, ejb, isem = idx_bufs[b % 2]
        _idx_wait(b, eib, ejb, isem)

        def _gs(ci, buf, gsem):
            pltpu.async_copy(he_hbm.at[ejb.at[ci, 0]], buf, gsem)

        def _gw(ci, buf, gsem):
            pltpu.make_async_copy(he_hbm.at[pl.ds(0, KC)], buf, gsem).wait()

        _gs(0, rows_a, gsa)

        def _pair(p, carry):
            a = 2 * p
            _gs(a + 1, rows_b, gsb)
            _gw(a, rows_a, gsa)
            _process(a, eib, rows_a)
            _gs(a + 2, rows_a, gsa)
            _gw(a + 1, rows_b, gsb)
            _process(a + 1, eib, rows_b)
            return carry
        lax.fori_loop(0, (CPB - 1) // 2, _pair, 0)
        _gw(CPB - 1, rows_a, gsa)
        _process(CPB - 1, eib, rows_a)

        if b + 2 < NB:
            _idx_start(b + 2, eib, ejb, isem)

    plsc.subcore_barrier()
    return cid, sid, base


def _gat_sc_body(he_hbm, sp_hbm, ei_hbm, ej_hbm, out_hbm,
                 ei_a, ej_a, ei_b, ej_b, sp_vm, rows_a, rows_b, w_vm, acc_sh,
                 gsa, gsb, isa, isb):
    cid, sid, base = _gat_core(he_hbm, sp_hbm, ei_hbm, ej_hbm,
                               ei_a, ej_a, ei_b, ej_b, sp_vm, rows_a, rows_b,
                               w_vm, acc_sh, gsa, gsb, isa, isb)
    pltpu.sync_copy(acc_sh.at[pl.ds(base, RPS)], out_hbm.at[cid, pl.ds(base, RPS)])


def _gat_sel_body(he_hbm, sp_hbm, ei_hbm, ej_hbm, idx_hbm, sel_hbm,
                  ei_a, ej_a, ei_b, ej_b, sp_vm, rows_a, rows_b, w_vm, acc_sh,
                  idx_vm, gsa, gsb, isa, isb):
    cid, sid, base = _gat_core(he_hbm, sp_hbm, ei_hbm, ej_hbm,
                               ei_a, ej_a, ei_b, ej_b, sp_vm, rows_a, rows_b,
                               w_vm, acc_sh, gsa, gsb, isa, isb)
    for h2 in range(N_IDX // (16 * IPW)):
        b2 = sid * (N_IDX // 16) + h2 * IPW
        pltpu.sync_copy(idx_hbm.at[pl.ds(b2, IPW)], idx_vm)
        pltpu.async_copy(acc_sh.at[idx_vm], rows_a.at[pl.ds(0, IPW)], gsa).wait()
        pltpu.sync_copy(rows_a.at[pl.ds(0, IPW)], sel_hbm.at[cid, pl.ds(b2, IPW)])


@functools.cache
def _gat_sc():
    return pl.kernel(
        _gat_sc_body,
        out_type=jax.ShapeDtypeStruct((2, NP, HE_W), jnp.float32),
        mesh=_mesh(),
        compiler_params=pltpu.CompilerParams(needs_layout_passes=False, use_tc_tiling_on_sc=False),
        scratch_types=[
            pltpu.VMEM((CPB, 1, KC), jnp.int32),
            pltpu.VMEM((CPB, 1, KC), jnp.int32),
            pltpu.VMEM((CPB, 1, KC), jnp.int32),
            pltpu.VMEM((CPB, 1, KC), jnp.int32),
            pltpu.VMEM((NP,), jnp.float32),
            pltpu.VMEM((KC, HE_W), jnp.float32),
            pltpu.VMEM((KC, HE_W), jnp.float32),
            pltpu.VMEM((KC,), jnp.float32),
            pltpu.VMEM_SHARED((NP, HE_W), jnp.float32),
            pltpu.SemaphoreType.DMA,
            pltpu.SemaphoreType.DMA,
            pltpu.SemaphoreType.DMA,
            pltpu.SemaphoreType.DMA,
        ],
    )


def _take_sc_body(y_hbm, idx_hbm, out_hbm, idx_vm, rows_vm, sem):
    cid = lax.axis_index("c")
    sid = lax.axis_index("s")
    base = (cid * 16 + sid) * IPW
    pltpu.sync_copy(idx_hbm.at[pl.ds(base, IPW)], idx_vm)
    pltpu.async_copy(y_hbm.at[idx_vm], rows_vm, sem).wait()
    pltpu.sync_copy(rows_vm, out_hbm.at[pl.ds(base, IPW)])


@functools.cache
def _gat_sel():
    return pl.kernel(
        _gat_sel_body,
        out_type=jax.ShapeDtypeStruct((2, N_IDX, HE_W), jnp.float32),
        mesh=_mesh(),
        compiler_params=pltpu.CompilerParams(needs_layout_passes=False, use_tc_tiling_on_sc=False),
        scratch_types=[
            pltpu.VMEM((CPB, 1, KC), jnp.int32),
            pltpu.VMEM((CPB, 1, KC), jnp.int32),
            pltpu.VMEM((CPB, 1, KC), jnp.int32),
            pltpu.VMEM((CPB, 1, KC), jnp.int32),
            pltpu.VMEM((NP,), jnp.float32),
            pltpu.VMEM((KC, HE_W), jnp.float32),
            pltpu.VMEM((KC, HE_W), jnp.float32),
            pltpu.VMEM((KC,), jnp.float32),
            pltpu.VMEM_SHARED((NP, HE_W), jnp.float32),
            pltpu.VMEM((IPW,), jnp.int32),
            pltpu.SemaphoreType.DMA,
            pltpu.SemaphoreType.DMA,
            pltpu.SemaphoreType.DMA,
            pltpu.SemaphoreType.DMA,
        ],
    )


@functools.cache
def _take_sc():
    return pl.kernel(
        _take_sc_body,
        out_type=jax.ShapeDtypeStruct((N_IDX, EMB), jnp.float32),
        mesh=_mesh(),
        compiler_params=pltpu.CompilerParams(needs_layout_passes=False, use_tc_tiling_on_sc=False),
        scratch_types=[
            pltpu.VMEM((IPW,), jnp.int32),
            pltpu.VMEM((IPW, EMB), jnp.float32),
            pltpu.SemaphoreType.DMA,
        ],
    )


def _a_pad(a1, a2):
    a = jnp.zeros((HID, 16), jnp.float32)
    return a.at[:, 0].set(a1).at[:, 1].set(a2)


def _bn_consts(gamma, beta, mean, var):
    sc = gamma / jnp.sqrt(var + 1e-5)
    sh = beta - mean * sc
    return sc.reshape(1, HID), sh.reshape(1, HID)


def kernel(features, edge_index, idx, W0, a1_0, a2_0, b0, gamma0, beta0,
           mean0, var0, W1, a1_1, a2_1, b1, gamma1, beta1, mean1, var1, Wd, bd):
    ei = edge_index[0].reshape(NWORK, NCH, 1, KC)
    ej = edge_index[1].reshape(NWORK, NCH, 1, KC)

    sc0, sh0 = _bn_consts(gamma0, beta0, mean0, var0)
    sc1, sh1 = _bn_consts(gamma1, beta1, mean1, var1)

    xp = jnp.pad(features, ((0, NP - N), (0, 0)))
    he0, sp0 = _encode(xp, W0, _a_pad(a1_0, a2_0))
    acc0 = _gat_sc()(he0, sp0.reshape(NP), ei, ej)
    he1, sp1 = _agg_encode(acc0, b0.reshape(1, HID), sc0, sh0, W1,
                           _a_pad(a1_1, a2_1))
    sel = _gat_sel()(he1, sp1.reshape(NP), ei, ej, idx)
    return _final_dense(sel, b1.reshape(1, HID), sc1, sh1, Wd, bd.reshape(1, EMB))
